# baseline (device time: 79940 ns/iter reference)
import jax
import jax.numpy as jnp
from jax import lax
from jax.experimental import pallas as pl
from jax.experimental.pallas import tpu as pltpu

N_DEV = 4
N_STREAMS = 4
N_HOPS = N_DEV - 1


def kernel(x):
    x2 = jnp.reshape(x, x.shape[1:])
    m, n = x2.shape
    c = m // N_DEV
    q = n // (2 * N_STREAMS)

    def body(x_ref, out_ref,
             sbuf_r, sbuf_l, rs_recv_r, rs_recv_l, ag_recv_r, ag_recv_l,
             rs_ssem_r, rs_rsem_r, rs_ssem_l, rs_rsem_l,
             ag_ssem_r, ag_rsem_r, ag_ssem_l, ag_rsem_l):
        my = lax.axis_index("i")
        left = lax.rem(my + (N_DEV - 1), N_DEV)
        right = lax.rem(my + 1, N_DEV)

        col_r = [st * q for st in range(N_STREAMS)]
        col_l = [(N_STREAMS + st) * q for st in range(N_STREAMS)]

        barrier_sem = pltpu.get_barrier_semaphore()
        for nbr in (left, right):
            pl.semaphore_signal(
                barrier_sem, inc=1,
                device_id=(nbr,), device_id_type=pl.DeviceIdType.MESH,
            )
        pl.semaphore_wait(barrier_sem, 2)

        def mk_rs(st, s, dirn):
            if dirn == "r":
                return pltpu.make_async_remote_copy(
                    src_ref=sbuf_r.at[st], dst_ref=rs_recv_r.at[st, s],
                    send_sem=rs_ssem_r.at[st, s], recv_sem=rs_rsem_r.at[st, s],
                    device_id=(right,), device_id_type=pl.DeviceIdType.MESH,
                )
            return pltpu.make_async_remote_copy(
                src_ref=sbuf_l.at[st], dst_ref=rs_recv_l.at[st, s],
                send_sem=rs_ssem_l.at[st, s], recv_sem=rs_rsem_l.at[st, s],
                device_id=(left,), device_id_type=pl.DeviceIdType.MESH,
            )

        def mk_ag(st, hh, dirn):
            if dirn == "r":
                src = sbuf_r.at[st] if hh == 0 else ag_recv_r.at[st, hh - 1]
                return pltpu.make_async_remote_copy(
                    src_ref=src, dst_ref=ag_recv_r.at[st, hh],
                    send_sem=ag_ssem_r.at[st, hh], recv_sem=ag_rsem_r.at[st, hh],
                    device_id=(right,), device_id_type=pl.DeviceIdType.MESH,
                )
            src = sbuf_l.at[st] if hh == 0 else ag_recv_l.at[st, hh - 1]
            return pltpu.make_async_remote_copy(
                src_ref=src, dst_ref=ag_recv_l.at[st, hh],
                send_sem=ag_ssem_l.at[st, hh], recv_sem=ag_rsem_l.at[st, hh],
                device_id=(left,), device_id_type=pl.DeviceIdType.MESH,
            )

        inflight = {}
        for st in range(N_STREAMS):
            sc = my
            sbuf_r[st] = x_ref[pl.ds(sc * c, c), pl.ds(col_r[st], q)]
            sbuf_l[st] = x_ref[pl.ds(sc * c, c), pl.ds(col_l[st], q)]
            for dirn in ("r", "l"):
                rdma = mk_rs(st, 0, dirn)
                rdma.start()
                inflight[(dirn, st)] = rdma

        for rnd in range(2 * N_HOPS):
            is_rs = rnd < N_HOPS
            s = rnd if is_rs else rnd - N_HOPS
            for st in range(N_STREAMS):
                for dirn in ("r", "l"):
                    rdma = inflight.pop((dirn, st))
                    rdma.wait()
                    sgn = -1 if dirn == "r" else 1
                    colw = col_r[st] if dirn == "r" else col_l[st]
                    sbuf = sbuf_r if dirn == "r" else sbuf_l
                    if is_rs:
                        rc = lax.rem(my + sgn * (s + 1) + N_DEV, N_DEV)
                        recv = rs_recv_r if dirn == "r" else rs_recv_l
                        acc = recv[st, s] + x_ref[pl.ds(rc * c, c),
                                                  pl.ds(colw, q)]
                        if s < N_HOPS - 1:
                            sbuf[st] = acc
                            nxt = mk_rs(st, s + 1, dirn)
                        else:
                            out_ref[pl.ds(rc * c, c), pl.ds(colw, q)] = acc
                            sbuf[st] = acc
                            nxt = mk_ag(st, 0, dirn)
                        nxt.start()
                        inflight[(dirn, st)] = nxt
                    else:
                        rc = lax.rem(my + sgn * s + N_DEV, N_DEV)
                        recv = ag_recv_r if dirn == "r" else ag_recv_l
                        if s < N_HOPS - 1:
                            nxt = mk_ag(st, s + 1, dirn)
                            nxt.start()
                            inflight[(dirn, st)] = nxt
                        out_ref[pl.ds(rc * c, c), pl.ds(colw, q)] = recv[st, s]

    return pl.pallas_call(
        body,
        out_shape=jax.ShapeDtypeStruct((m, n), x2.dtype),
        in_specs=[pl.BlockSpec(memory_space=pltpu.VMEM)],
        out_specs=pl.BlockSpec(memory_space=pltpu.VMEM),
        scratch_shapes=[
            pltpu.VMEM((N_STREAMS, c, q), x2.dtype),
            pltpu.VMEM((N_STREAMS, c, q), x2.dtype),
            pltpu.VMEM((N_STREAMS, N_HOPS, c, q), x2.dtype),
            pltpu.VMEM((N_STREAMS, N_HOPS, c, q), x2.dtype),
            pltpu.VMEM((N_STREAMS, N_HOPS, c, q), x2.dtype),
            pltpu.VMEM((N_STREAMS, N_HOPS, c, q), x2.dtype),
            pltpu.SemaphoreType.DMA((N_STREAMS, N_HOPS)),
            pltpu.SemaphoreType.DMA((N_STREAMS, N_HOPS)),
            pltpu.SemaphoreType.DMA((N_STREAMS, N_HOPS)),
            pltpu.SemaphoreType.DMA((N_STREAMS, N_HOPS)),
            pltpu.SemaphoreType.DMA((N_STREAMS, N_HOPS)),
            pltpu.SemaphoreType.DMA((N_STREAMS, N_HOPS)),
            pltpu.SemaphoreType.DMA((N_STREAMS, N_HOPS)),
            pltpu.SemaphoreType.DMA((N_STREAMS, N_HOPS)),
        ],
        compiler_params=pltpu.CompilerParams(collective_id=0),
    )(x2)


# device time: 79474 ns/iter; 1.0059x vs baseline; 1.0059x over previous
import jax
import jax.numpy as jnp
from jax import lax
from jax.experimental import pallas as pl
from jax.experimental.pallas import tpu as pltpu

N_DEV = 4
N_STREAMS = 2
N_HOPS = N_DEV - 1


def kernel(x):
    _, m, n = x.shape
    c = m // N_DEV
    q = n // (2 * N_STREAMS)

    def body(x_ref, out_ref,
             sbuf_r, sbuf_l, rs_recv_r, rs_recv_l, ag_recv_r, ag_recv_l,
             rs_ssem_r, rs_rsem_r, rs_ssem_l, rs_rsem_l,
             ag_ssem_r, ag_rsem_r, ag_ssem_l, ag_rsem_l):
        my = lax.axis_index("i")
        left = lax.rem(my + (N_DEV - 1), N_DEV)
        right = lax.rem(my + 1, N_DEV)

        col_r = [st * q for st in range(N_STREAMS)]
        col_l = [(N_STREAMS + st) * q for st in range(N_STREAMS)]

        barrier_sem = pltpu.get_barrier_semaphore()
        for nbr in (left, right):
            pl.semaphore_signal(
                barrier_sem, inc=1,
                device_id=(nbr,), device_id_type=pl.DeviceIdType.MESH,
            )
        pl.semaphore_wait(barrier_sem, 2)

        def mk_rs(st, s, dirn):
            if dirn == "r":
                return pltpu.make_async_remote_copy(
                    src_ref=sbuf_r.at[st], dst_ref=rs_recv_r.at[st, s],
                    send_sem=rs_ssem_r.at[st, s], recv_sem=rs_rsem_r.at[st, s],
                    device_id=(right,), device_id_type=pl.DeviceIdType.MESH,
                )
            return pltpu.make_async_remote_copy(
                src_ref=sbuf_l.at[st], dst_ref=rs_recv_l.at[st, s],
                send_sem=rs_ssem_l.at[st, s], recv_sem=rs_rsem_l.at[st, s],
                device_id=(left,), device_id_type=pl.DeviceIdType.MESH,
            )

        def mk_ag(st, hh, dirn):
            if dirn == "r":
                src = sbuf_r.at[st] if hh == 0 else ag_recv_r.at[st, hh - 1]
                return pltpu.make_async_remote_copy(
                    src_ref=src, dst_ref=ag_recv_r.at[st, hh],
                    send_sem=ag_ssem_r.at[st, hh], recv_sem=ag_rsem_r.at[st, hh],
                    device_id=(right,), device_id_type=pl.DeviceIdType.MESH,
                )
            src = sbuf_l.at[st] if hh == 0 else ag_recv_l.at[st, hh - 1]
            return pltpu.make_async_remote_copy(
                src_ref=src, dst_ref=ag_recv_l.at[st, hh],
                send_sem=ag_ssem_l.at[st, hh], recv_sem=ag_rsem_l.at[st, hh],
                device_id=(left,), device_id_type=pl.DeviceIdType.MESH,
            )

        inflight = {}
        for st in range(N_STREAMS):
            sc = my
            sbuf_r[st] = x_ref[0, pl.ds(sc * c, c), pl.ds(col_r[st], q)]
            sbuf_l[st] = x_ref[0, pl.ds(sc * c, c), pl.ds(col_l[st], q)]
            for dirn in ("r", "l"):
                rdma = mk_rs(st, 0, dirn)
                rdma.start()
                inflight[(dirn, st)] = rdma

        for rnd in range(2 * N_HOPS):
            is_rs = rnd < N_HOPS
            s = rnd if is_rs else rnd - N_HOPS
            for st in range(N_STREAMS):
                for dirn in ("r", "l"):
                    rdma = inflight.pop((dirn, st))
                    rdma.wait()
                    sgn = -1 if dirn == "r" else 1
                    colw = col_r[st] if dirn == "r" else col_l[st]
                    sbuf = sbuf_r if dirn == "r" else sbuf_l
                    if is_rs:
                        rc = lax.rem(my + sgn * (s + 1) + N_DEV, N_DEV)
                        recv = rs_recv_r if dirn == "r" else rs_recv_l
                        acc = recv[st, s] + x_ref[0, pl.ds(rc * c, c),
                                                  pl.ds(colw, q)]
                        if s < N_HOPS - 1:
                            sbuf[st] = acc
                            nxt = mk_rs(st, s + 1, dirn)
                        else:
                            out_ref[pl.ds(rc * c, c), pl.ds(colw, q)] = acc
                            sbuf[st] = acc
                            nxt = mk_ag(st, 0, dirn)
                        nxt.start()
                        inflight[(dirn, st)] = nxt
                    else:
                        rc = lax.rem(my + sgn * s + N_DEV, N_DEV)
                        recv = ag_recv_r if dirn == "r" else ag_recv_l
                        if s < N_HOPS - 1:
                            nxt = mk_ag(st, s + 1, dirn)
                            nxt.start()
                            inflight[(dirn, st)] = nxt
                        out_ref[pl.ds(rc * c, c), pl.ds(colw, q)] = recv[st, s]

    return pl.pallas_call(
        body,
        out_shape=jax.ShapeDtypeStruct((m, n), x.dtype),
        in_specs=[pl.BlockSpec(memory_space=pltpu.VMEM)],
        out_specs=pl.BlockSpec(memory_space=pltpu.VMEM),
        scratch_shapes=[
            pltpu.VMEM((N_STREAMS, c, q), x.dtype),
            pltpu.VMEM((N_STREAMS, c, q), x.dtype),
            pltpu.VMEM((N_STREAMS, N_HOPS, c, q), x.dtype),
            pltpu.VMEM((N_STREAMS, N_HOPS, c, q), x.dtype),
            pltpu.VMEM((N_STREAMS, N_HOPS, c, q), x.dtype),
            pltpu.VMEM((N_STREAMS, N_HOPS, c, q), x.dtype),
            pltpu.SemaphoreType.DMA((N_STREAMS, N_HOPS)),
            pltpu.SemaphoreType.DMA((N_STREAMS, N_HOPS)),
            pltpu.SemaphoreType.DMA((N_STREAMS, N_HOPS)),
            pltpu.SemaphoreType.DMA((N_STREAMS, N_HOPS)),
            pltpu.SemaphoreType.DMA((N_STREAMS, N_HOPS)),
            pltpu.SemaphoreType.DMA((N_STREAMS, N_HOPS)),
            pltpu.SemaphoreType.DMA((N_STREAMS, N_HOPS)),
            pltpu.SemaphoreType.DMA((N_STREAMS, N_HOPS)),
        ],
        compiler_params=pltpu.CompilerParams(collective_id=0),
    )(x)
